# baseline (speedup 1.0000x reference)
import jax
import jax.numpy as jnp
from jax.experimental import pallas as pl
from jax.experimental.pallas import tpu as pltpu

_EPS = 1e-8
_STREAMS = 16
_FTILE = 128
_NBUF = 6
_LROWS = _FTILE // 128


def _norm_protos(ca):
    ca_sq = jnp.sum(ca * ca, axis=1, keepdims=True)
    ca_inv = 1.0 / jnp.maximum(jnp.sqrt(ca_sq), _EPS)
    return ca * ca_inv


def _cos_tile(x, ca_unit, ones_d):
    raw = jax.lax.dot_general(
        x, ca_unit, (((1,), (1,)), ((), ())),
        preferred_element_type=jnp.float32)
    row_sq = jax.lax.dot_general(
        x * x, ones_d, (((1,), (1,)), ((), ())),
        preferred_element_type=jnp.float32)
    inv = 1.0 / jnp.maximum(jnp.sqrt(row_sq), _EPS)
    return raw * inv


def _pipeline_body(x_hbm, lab_ref, ca_ref, o_hbm,
                   xbuf, obuf, in_sems, out_sems):
    nsteps = x_hbm.shape[1] // _FTILE
    ca = ca_ref[...]
    k = ca.shape[0]

    inf_mask = jnp.where(jnp.isinf(ca), 1.0, 0.0)
    ones_row = jnp.ones((1, ca.shape[1]), jnp.float32)
    has_inf = jax.lax.dot_general(
        ones_row, inf_mask, (((1,), (1,)), ((), ())),
        preferred_element_type=jnp.float32) > 0.0
    col_has_inf = jnp.max(inf_mask, axis=1, keepdims=True) > 0.0
    ca_unit = _norm_protos(jnp.where(col_has_inf, 0.0, ca))

    tmod = jax.lax.broadcasted_iota(jnp.int32, (_FTILE, 128), 0)
    lane = jax.lax.broadcasted_iota(jnp.int32, (_FTILE, 128), 1)
    pick = jnp.where(lane == tmod, 1.0, 0.0)
    kidx = jax.lax.broadcasted_iota(jnp.int32, (_FTILE, k), 1)
    ones_d = jnp.ones((1, ca.shape[1]), jnp.float32)

    def in_copies(j):
        slot = j % _NBUF
        return [pltpu.make_async_copy(
            x_hbm.at[s, pl.ds(j * _FTILE, _FTILE), :],
            xbuf.at[slot, s],
            in_sems.at[slot, s]) for s in range(_STREAMS)]

    def out_copy(j):
        return pltpu.make_async_copy(
            obuf.at[j % 2],
            o_hbm.at[:, pl.ds(j * _FTILE, _FTILE), :],
            out_sems.at[j % 2])

    for jj in range(5):
        for c in in_copies(jj):
            c.start()

    for j in range(nsteps):
        if j + 5 < nsteps:
            for c in in_copies(j + 5):
                c.start()
        for c in in_copies(j):
            c.wait()
        if j >= 2:
            out_copy(j - 2).wait()
        slot = j % _NBUF
        for s in range(_STREAMS):
            cos = _cos_tile(xbuf[slot, s], ca_unit, ones_d)
            labl = lab_ref[s, j].astype(jnp.float32)
            labs = jax.lax.dot_general(
                pick, labl, (((1,), (1,)), ((), ())),
                preferred_element_type=jnp.float32)
            uninit = jnp.where(labs.astype(jnp.int32) == kidx, 1.0, -1.0)
            obuf[j % 2, s] = jnp.where(has_inf, uninit, cos)
        out_copy(j).start()

    out_copy(nsteps - 2).wait()
    out_copy(nsteps - 1).wait()


def kernel(inputs, labels, class_avgs):
    b, t, d = inputs.shape
    k = class_avgs.shape[0]
    rows = (b * t) // _STREAMS
    nsteps = rows // _FTILE
    x3 = inputs.reshape(_STREAMS, rows, d)
    labs4 = labels.astype(jnp.int32).reshape(_STREAMS, nsteps, _LROWS, 128)

    out = pl.pallas_call(
        _pipeline_body,
        in_specs=[
            pl.BlockSpec(memory_space=pl.ANY),
            pl.BlockSpec(memory_space=pltpu.MemorySpace.VMEM),
            pl.BlockSpec(memory_space=pltpu.MemorySpace.VMEM),
        ],
        out_specs=pl.BlockSpec(memory_space=pl.ANY),
        out_shape=jax.ShapeDtypeStruct((_STREAMS, rows, k), jnp.float32),
        scratch_shapes=[
            pltpu.VMEM((_NBUF, _STREAMS, _FTILE, d), jnp.float32),
            pltpu.VMEM((2, _STREAMS, _FTILE, k), jnp.float32),
            pltpu.SemaphoreType.DMA((_NBUF, _STREAMS)),
            pltpu.SemaphoreType.DMA((2,)),
        ],
    )(x3, labs4, class_avgs)
    return out.reshape(b, t, k)


# speedup vs baseline: 1.0175x; 1.0175x over previous
import jax
import jax.numpy as jnp
from jax.experimental import pallas as pl
from jax.experimental.pallas import tpu as pltpu

_EPS = 1e-8
_STREAMS = 8
_FTILE = 128
_NBUF = 6
_LROWS = _FTILE // 128


def _norm_protos(ca):
    ca_sq = jnp.sum(ca * ca, axis=1, keepdims=True)
    ca_inv = 1.0 / jnp.maximum(jnp.sqrt(ca_sq), _EPS)
    return ca * ca_inv


def _cos_tile(x, ca_unit, ones_d):
    raw = jax.lax.dot_general(
        x, ca_unit, (((1,), (1,)), ((), ())),
        preferred_element_type=jnp.float32)
    row_sq = jax.lax.dot_general(
        x * x, ones_d, (((1,), (1,)), ((), ())),
        preferred_element_type=jnp.float32)
    inv = 1.0 / jnp.maximum(jnp.sqrt(row_sq), _EPS)
    return raw * inv


def _pipeline_body(x_hbm, lab_ref, ca_ref, o_hbm,
                   xbuf, obuf, in_sems, out_sems):
    nsteps = x_hbm.shape[1] // _FTILE
    ca = ca_ref[...]
    k = ca.shape[0]

    inf_mask = jnp.where(jnp.isinf(ca), 1.0, 0.0)
    ones_row = jnp.ones((1, ca.shape[1]), jnp.float32)
    has_inf = jax.lax.dot_general(
        ones_row, inf_mask, (((1,), (1,)), ((), ())),
        preferred_element_type=jnp.float32) > 0.0
    col_has_inf = jnp.max(inf_mask, axis=1, keepdims=True) > 0.0
    ca_unit = _norm_protos(jnp.where(col_has_inf, 0.0, ca))

    tmod = jax.lax.broadcasted_iota(jnp.int32, (_FTILE, 128), 0)
    lane = jax.lax.broadcasted_iota(jnp.int32, (_FTILE, 128), 1)
    pick = jnp.where(lane == tmod, 1.0, 0.0)
    kidx = jax.lax.broadcasted_iota(jnp.int32, (_FTILE, k), 1)
    ones_d = jnp.ones((1, ca.shape[1]), jnp.float32)

    def in_copies(j):
        slot = j % _NBUF
        return [pltpu.make_async_copy(
            x_hbm.at[s, pl.ds(j * _FTILE, _FTILE), :],
            xbuf.at[slot, s],
            in_sems.at[slot, s]) for s in range(_STREAMS)]

    def out_copy(j):
        return pltpu.make_async_copy(
            obuf.at[j % 2],
            o_hbm.at[:, pl.ds(j * _FTILE, _FTILE), :],
            out_sems.at[j % 2])

    for jj in range(5):
        for c in in_copies(jj):
            c.start()

    for j in range(nsteps):
        if j + 5 < nsteps:
            for c in in_copies(j + 5):
                c.start()
        for c in in_copies(j):
            c.wait()
        if j >= 2:
            out_copy(j - 2).wait()
        slot = j % _NBUF
        for s in range(_STREAMS):
            cos = _cos_tile(xbuf[slot, s], ca_unit, ones_d)
            labl = lab_ref[s, j].astype(jnp.float32)
            labs = jax.lax.dot_general(
                pick, labl, (((1,), (1,)), ((), ())),
                preferred_element_type=jnp.float32)
            uninit = jnp.where(labs.astype(jnp.int32) == kidx, 1.0, -1.0)
            obuf[j % 2, s] = jnp.where(has_inf, uninit, cos)
        out_copy(j).start()

    out_copy(nsteps - 2).wait()
    out_copy(nsteps - 1).wait()


def kernel(inputs, labels, class_avgs):
    b, t, d = inputs.shape
    k = class_avgs.shape[0]
    rows = (b * t) // _STREAMS
    nsteps = rows // _FTILE
    x3 = inputs.reshape(_STREAMS, rows, d)
    labs4 = labels.astype(jnp.int32).reshape(_STREAMS, nsteps, _LROWS, 128)

    out = pl.pallas_call(
        _pipeline_body,
        in_specs=[
            pl.BlockSpec(memory_space=pl.ANY),
            pl.BlockSpec(memory_space=pltpu.MemorySpace.VMEM),
            pl.BlockSpec(memory_space=pltpu.MemorySpace.VMEM),
        ],
        out_specs=pl.BlockSpec(memory_space=pl.ANY),
        out_shape=jax.ShapeDtypeStruct((_STREAMS, rows, k), jnp.float32),
        scratch_shapes=[
            pltpu.VMEM((_NBUF, _STREAMS, _FTILE, d), jnp.float32),
            pltpu.VMEM((2, _STREAMS, _FTILE, k), jnp.float32),
            pltpu.SemaphoreType.DMA((_NBUF, _STREAMS)),
            pltpu.SemaphoreType.DMA((2,)),
        ],
    )(x3, labs4, class_avgs)
    return out.reshape(b, t, k)
